# Initial kernel scaffold; baseline (speedup 1.0000x reference)
import jax
import jax.numpy as jnp
from jax.experimental import pallas as pl


def kernel(inputs, embedding):
    raise NotImplementedError("write your pallas kernel here")


# speedup vs baseline: 1.7995x; 1.7995x over previous
import jax
import jax.numpy as jnp
from jax.experimental import pallas as pl

_B = 16
_C = 64
_HW = 1024
_K = 1024
_T = 512


def _vq_block_kernel(x_ref, e_ref, out_ref, loss_ref):
    x = x_ref[0]
    e = e_ref[...]

    mm = jax.lax.dot_general(
        x, e, (((0,), (1,)), ((), ())),
        preferred_element_type=jnp.float32)

    xnorm = jnp.sum(x * x, axis=0)
    enorm = jnp.sum(e * e, axis=1)
    d = (xnorm[:, None] + enorm[None, :]) - 2.0 * mm

    dmin = jnp.min(d, axis=1)
    iota = jax.lax.broadcasted_iota(jnp.int32, (_T, _K), 1)
    idx = jnp.min(jnp.where(d == dmin[:, None], iota, _K), axis=1)

    onehot = (iota == idx[:, None]).astype(jnp.float32)
    q = jnp.dot(onehot, e, preferred_element_type=jnp.float32)

    qt = q.T
    out_ref[0] = qt

    diff = qt - x
    blk = jnp.sum(diff * diff).reshape(1, 1)

    @pl.when((pl.program_id(0) == 0) & (pl.program_id(1) == 0))
    def _():
        loss_ref[...] = jnp.zeros_like(loss_ref)

    loss_ref[...] += blk


def kernel(inputs, embedding):
    x3 = inputs.reshape(_B, _C, _HW)
    n_t = _HW // _T
    out, loss_sum = pl.pallas_call(
        _vq_block_kernel,
        grid=(_B, n_t),
        in_specs=[
            pl.BlockSpec((1, _C, _T), lambda b, t: (b, 0, t)),
            pl.BlockSpec((_K, _C), lambda b, t: (0, 0)),
        ],
        out_specs=[
            pl.BlockSpec((1, _C, _T), lambda b, t: (b, 0, t)),
            pl.BlockSpec((1, 1), lambda b, t: (0, 0)),
        ],
        out_shape=[
            jax.ShapeDtypeStruct((_B, _C, _HW), jnp.float32),
            jax.ShapeDtypeStruct((1, 1), jnp.float32),
        ],
    )(x3, embedding)

    m = loss_sum[0, 0] / (_B * _HW * _C)
    loss = m + 0.25 * m
    return out.reshape(_B, _C, 32, 32), loss


# speedup vs baseline: 2.0196x; 1.1223x over previous
import jax
import jax.numpy as jnp
from jax.experimental import pallas as pl
from jax.experimental.pallas import tpu as pltpu

_B = 16
_C = 64
_HW = 1024
_K = 1024
_T = 1024


def _vq_block_kernel(x_ref, e_ref, out_ref, loss_ref, en_ref):
    x = x_ref[0]
    e = e_ref[...]

    @pl.when(pl.program_id(0) == 0)
    def _():
        en_ref[...] = jnp.sum(e * e, axis=1)[None, :]
        loss_ref[...] = jnp.zeros_like(loss_ref)

    mm = jax.lax.dot_general(
        x, e, (((0,), (1,)), ((), ())),
        preferred_element_type=jnp.float32)

    xnorm = jnp.sum(x * x, axis=0)
    d = (xnorm[:, None] + en_ref[...]) - 2.0 * mm

    dmin = jnp.min(d, axis=1)
    iota = jax.lax.broadcasted_iota(jnp.int32, (_T, _K), 1)
    idx = jnp.min(jnp.where(d == dmin[:, None], iota, _K), axis=1)

    onehot = (iota == idx[:, None]).astype(jnp.float32)
    q = jnp.dot(onehot, e, preferred_element_type=jnp.float32)

    qt = q.T
    out_ref[0] = qt

    diff = qt - x
    loss_ref[...] += jnp.sum(diff * diff).reshape(1, 1)


def kernel(inputs, embedding):
    x3 = inputs.reshape(_B, _C, _HW)
    n_t = _HW // _T
    out, loss_sum = pl.pallas_call(
        _vq_block_kernel,
        grid=(_B * n_t,),
        in_specs=[
            pl.BlockSpec((1, _C, _T), lambda i: (i // n_t, 0, i % n_t)),
            pl.BlockSpec((_K, _C), lambda i: (0, 0)),
        ],
        out_specs=[
            pl.BlockSpec((1, _C, _T), lambda i: (i // n_t, 0, i % n_t)),
            pl.BlockSpec((1, 1), lambda i: (0, 0)),
        ],
        out_shape=[
            jax.ShapeDtypeStruct((_B, _C, _HW), jnp.float32),
            jax.ShapeDtypeStruct((1, 1), jnp.float32),
        ],
        scratch_shapes=[pltpu.VMEM((1, _K), jnp.float32)],
    )(x3, embedding)

    m = loss_sum[0, 0] / (_B * _HW * _C)
    loss = m + 0.25 * m
    return out.reshape(_B, _C, 32, 32), loss


# speedup vs baseline: 2.0738x; 1.0268x over previous
import jax
import jax.numpy as jnp
from jax.experimental import pallas as pl
from jax.experimental.pallas import tpu as pltpu

_B = 16
_C = 64
_HW = 1024
_K = 1024
_T = 1024


def _vq_block_kernel(x_ref, e_ref, out_ref, loss_ref, en_ref):
    x = x_ref[0]
    e = e_ref[...]

    @pl.when(pl.program_id(0) == 0)
    def _():
        en_ref[...] = jnp.sum(e * e, axis=1)[None, :]
        loss_ref[...] = jnp.zeros_like(loss_ref)

    mm = jax.lax.dot_general(
        x, e, (((0,), (1,)), ((), ())),
        preferred_element_type=jnp.float32)

    xnorm = jnp.sum(x * x, axis=0)
    d = (xnorm[:, None] + en_ref[...]) - 2.0 * mm

    dmin = jnp.min(d, axis=1)
    iota = jax.lax.broadcasted_iota(jnp.int32, (_T, _K), 1)
    idx = jnp.min(jnp.where(d == dmin[:, None], iota, _K), axis=1)

    onehot = (iota == idx[:, None]).astype(jnp.float32)
    q = jnp.dot(onehot, e, preferred_element_type=jnp.float32)

    out_ref[0] = q.T

    loss_ref[...] += (jnp.sum(xnorm) + jnp.sum(dmin)).reshape(1, 1)


def kernel(inputs, embedding):
    x3 = inputs.reshape(_B, _C, _HW)
    n_t = _HW // _T
    out, loss_sum = pl.pallas_call(
        _vq_block_kernel,
        grid=(_B * n_t,),
        in_specs=[
            pl.BlockSpec((1, _C, _T), lambda i: (i // n_t, 0, i % n_t)),
            pl.BlockSpec((_K, _C), lambda i: (0, 0)),
        ],
        out_specs=[
            pl.BlockSpec((1, _C, _T), lambda i: (i // n_t, 0, i % n_t)),
            pl.BlockSpec((1, 1), lambda i: (0, 0)),
        ],
        out_shape=[
            jax.ShapeDtypeStruct((_B, _C, _HW), jnp.float32),
            jax.ShapeDtypeStruct((1, 1), jnp.float32),
        ],
        scratch_shapes=[pltpu.VMEM((1, _K), jnp.float32)],
    )(x3, embedding)

    m = loss_sum[0, 0] / (_B * _HW * _C)
    loss = m + 0.25 * m
    return out.reshape(_B, _C, 32, 32), loss


# speedup vs baseline: 2.0743x; 1.0003x over previous
import jax
import jax.numpy as jnp
from jax.experimental import pallas as pl
from jax.experimental.pallas import tpu as pltpu

_B = 16
_C = 64
_HW = 1024
_K = 1024
_T = 1024


def _vq_block_kernel(x_ref, e_ref, out_ref, loss_ref, en_ref):
    x = x_ref[0]
    e = e_ref[...]

    @pl.when(pl.program_id(0) == 0)
    def _():
        en_ref[...] = jnp.sum(e * e, axis=1)[None, :]
        loss_ref[...] = jnp.zeros_like(loss_ref)

    mm = jax.lax.dot_general(
        x, e, (((0,), (1,)), ((), ())),
        preferred_element_type=jnp.float32)

    xnorm = jnp.sum(x * x, axis=0)
    d = (xnorm[:, None] + en_ref[...]) - 2.0 * mm

    dmin = jnp.min(d, axis=1)
    iota = jax.lax.broadcasted_iota(jnp.int32, (_T, _K), 1)
    idx = jnp.min(jnp.where(d == dmin[:, None], iota, _K), axis=1)

    onehot = (iota == idx[:, None]).astype(jnp.float32)
    q = jnp.dot(onehot, e, preferred_element_type=jnp.float32)

    out_ref[0] = q.T

    loss_ref[...] += jnp.sum(dmin).reshape(1, 1)


def kernel(inputs, embedding):
    x3 = inputs.reshape(_B, _C, _HW)
    n_t = _HW // _T
    out, loss_sum = pl.pallas_call(
        _vq_block_kernel,
        grid=(_B * n_t,),
        in_specs=[
            pl.BlockSpec((1, _C, _T), lambda i: (i // n_t, 0, i % n_t)),
            pl.BlockSpec((_K, _C), lambda i: (0, 0)),
        ],
        out_specs=[
            pl.BlockSpec((1, _C, _T), lambda i: (i // n_t, 0, i % n_t)),
            pl.BlockSpec((1, 1), lambda i: (0, 0)),
        ],
        out_shape=[
            jax.ShapeDtypeStruct((_B, _C, _HW), jnp.float32),
            jax.ShapeDtypeStruct((1, 1), jnp.float32),
        ],
        scratch_shapes=[pltpu.VMEM((1, _K), jnp.float32)],
    )(x3, embedding)

    m = loss_sum[0, 0] / (_B * _HW * _C)
    loss = m + 0.25 * m
    return out.reshape(_B, _C, 32, 32), loss
